# baseline (speedup 1.0000x reference)
import functools
import math

import jax
import jax.numpy as jnp
from jax import lax
from jax.experimental import pallas as pl
from jax.experimental.pallas import tpu as pltpu
from jax.experimental.pallas import tpu_sc as plsc

VOCAB = 1000000
D = 64
SCALE = math.sqrt(D)

NC = 2
NS = 16
NW = NC * NS

B = 16384 * 20
B_PER_W = B // NW
CHUNK = 128
NCH = B_PER_W // CHUNK
LPR = D // 16
NBUF = 4
DP = 128


@functools.partial(
    pl.kernel,
    mesh=plsc.VectorSubcoreMesh(core_axis_name="c", subcore_axis_name="s"),
    out_type=jax.ShapeDtypeStruct((B, D), jnp.float32),
    scratch_types=[
        pltpu.VMEM((NCH, CHUNK), jnp.int32),
        pltpu.VMEM((NBUF, CHUNK, DP), jnp.float32),
        pltpu.VMEM((NBUF, CHUNK, D), jnp.float32),
        pltpu.SemaphoreType.DMA((NBUF,)),
        pltpu.SemaphoreType.DMA((NBUF,)),
    ],
    compiler_params=pltpu.CompilerParams(use_tc_tiling_on_sc=False),
)
def _embed_kernel(idx_hbm, table_hbm, out_hbm, idx_v, gbuf, sbuf, gsem, ssem):
    cid = lax.axis_index("c")
    sid = lax.axis_index("s")
    wid = sid * NC + cid
    pltpu.sync_copy(idx_hbm.at[wid], idx_v)
    base = wid * B_PER_W

    def start_gather(c, b):
        pltpu.async_copy(table_hbm.at[idx_v.at[c]], gbuf.at[b], gsem.at[b])

    def wait_gather(c, b):
        pltpu.make_async_copy(table_hbm.at[idx_v.at[c]], gbuf.at[b],
                              gsem.at[b]).wait()

    def out_slot(c):
        return out_hbm.at[pl.ds(base + c * CHUNK, CHUNK)]

    def start_scatter(c, b):
        pltpu.async_copy(sbuf.at[b], out_slot(c), ssem.at[b])

    def wait_scatter(c, b):
        pltpu.make_async_copy(sbuf.at[b], out_slot(c), ssem.at[b]).wait()

    def scale_chunk(b):
        def scale_row(r, carry):
            for j in range(LPR):
                sl = pl.ds(j * 16, 16)
                sbuf[b, r, sl] = gbuf[b, r, sl] * SCALE
            return carry

        lax.fori_loop(0, CHUNK, scale_row, 0)


    for b in range(NBUF):
        start_gather(b, b)

    for b in range(NBUF):
        wait_gather(b, b)
        scale_chunk(b)
        start_gather(b + NBUF, b)
        start_scatter(b, b)

    @pl.loop(NBUF, NCH - NBUF, step=NBUF)
    def _steady(ci):
        for b in range(NBUF):
            c = ci + b
            wait_gather(c, b)
            wait_scatter(c - NBUF, b)
            scale_chunk(b)
            start_gather(c + NBUF, b)
            start_scatter(c, b)

    for b in range(NBUF):
        c = NCH - NBUF + b
        wait_gather(c, b)
        wait_scatter(c - NBUF, b)
        scale_chunk(b)
        start_scatter(c, b)

    for b in range(NBUF):
        wait_scatter(NCH - NBUF + b, b)


def kernel(x, table):
    idx = x.astype(jnp.int32).reshape(NW, NCH, CHUNK)
    table128 = jnp.pad(table, ((0, 0), (0, D)))
    out = _embed_kernel(idx, table128)
    return out.reshape(x.shape[0], x.shape[1], D)


# speedup vs baseline: 1.0887x; 1.0887x over previous
import functools
import math

import jax
import jax.numpy as jnp
from jax import lax
from jax.experimental import pallas as pl
from jax.experimental.pallas import tpu as pltpu
from jax.experimental.pallas import tpu_sc as plsc

VOCAB = 1000000
D = 64
SCALE = math.sqrt(D)

NC = 2
NS = 16
NW = NC * NS

NB = 16384
NP = 20
BPW = NB // NW
HALF = BPW // 2
NSTEP = NP * 2
FT = D // 8
BT = NB // 128
BTW = BPW // 128


@functools.partial(
    pl.kernel,
    mesh=plsc.VectorSubcoreMesh(core_axis_name="c", subcore_axis_name="s"),
    out_type=jax.ShapeDtypeStruct((NP, FT, BT, 8, 128), jnp.float32),
    scratch_types=[
        pltpu.VMEM((NP, BPW), jnp.int32),
        pltpu.VMEM((2, HALF, D), jnp.float32),
        pltpu.VMEM((2, FT, 2, 8, 128), jnp.float32),
        pltpu.SemaphoreType.DMA((2,)),
        pltpu.SemaphoreType.DMA((2,)),
    ],
    compiler_params=pltpu.CompilerParams(
        use_tc_tiling_on_sc=False, needs_layout_passes=False
    ),
)
def _embed_kernel(xt_hbm, table_hbm, out_hbm, idx_v, gbuf, tbuf, gsem, ssem):
    cid = lax.axis_index("c")
    sid = lax.axis_index("s")
    wid = sid * NC + cid
    b0 = wid * BPW

    for p in range(NP):
        pltpu.sync_copy(xt_hbm.at[p, pl.ds(b0, BPW)], idx_v.at[p])

    iota = lax.iota(jnp.int32, 16)

    def gathers(s, b, start):
        p = s // 2
        h = s % 2
        for jj in range(2):
            idxref = idx_v.at[p, pl.ds(h * HALF + jj * 128, 128)]
            cp = (pltpu.async_copy if start else pltpu.make_async_copy)(
                table_hbm.at[idxref],
                gbuf.at[b, pl.ds(jj * 128, 128)],
                gsem.at[b],
            )
            if not start:
                cp.wait()

    def scatters(s, b, start):
        p = s // 2
        h = s % 2
        bt0 = wid * BTW + h * 2
        for ft in range(FT):
            cp = (pltpu.async_copy if start else pltpu.make_async_copy)(
                tbuf.at[b, ft],
                out_hbm.at[p, ft, pl.ds(bt0, 2)],
                ssem.at[b],
            )
            if not start:
                cp.wait()

    def transpose_scale(b):
        @plsc.parallel_loop(0, D, unroll=2)
        def _f(f):
            ft = f // 8
            fi = f % 8
            cols = jnp.full((16,), 0, jnp.int32) + f
            for k in range(16):
                rows = iota + (k * 16)
                vals = plsc.load_gather(gbuf.at[b], [rows, cols])
                tbuf[b, ft, k // 8, fi, pl.ds((k % 8) * 16, 16)] = vals * SCALE

    def step(s, b, first, last):
        gathers(s, b, start=False)
        if not last:
            gathers(s + 1, 1 - b, start=True)
        if not first:
            scatters(s - 2, b, start=False)
        transpose_scale(b)
        scatters(s, b, start=True)

    gathers(0, 0, start=True)
    step(0, 0, first=True, last=False)
    step(1, 1, first=True, last=False)

    @pl.loop(2, NSTEP - 2, step=2)
    def _steady(s):
        step(s, 0, first=False, last=False)
        step(s + 1, 1, first=False, last=False)

    step(NSTEP - 2, 0, first=False, last=False)
    step(NSTEP - 1, 1, first=False, last=True)
    scatters(NSTEP - 2, 0, start=False)
    scatters(NSTEP - 1, 1, start=False)


def kernel(x, table):
    xt = jnp.transpose(x.astype(jnp.int32))
    out5d = _embed_kernel(xt, table)
    return out5d.transpose(2, 4, 0, 1, 3).reshape(NB, NP, D)


# speedup vs baseline: 1.3982x; 1.2842x over previous
import functools
import math

import jax
import jax.numpy as jnp
from jax import lax
from jax.experimental import pallas as pl
from jax.experimental.pallas import tpu as pltpu
from jax.experimental.pallas import tpu_sc as plsc

VOCAB = 1000000
D = 64
SCALE = math.sqrt(D)

NC = 2
NS = 16
NW = NC * NS

NB = 16384
NP = 20
BPW = NB // NW
HALF = BPW // 2
NSTEP = NP * 2
FT = D // 8
BT = NB // 128
BTW = BPW // 128


@functools.partial(
    pl.kernel,
    mesh=plsc.VectorSubcoreMesh(core_axis_name="c", subcore_axis_name="s"),
    out_type=jax.ShapeDtypeStruct((NP, FT, BT, 8, 128), jnp.float32),
    scratch_types=[
        pltpu.VMEM((NP, BPW), jnp.int32),
        pltpu.VMEM((2, HALF, D), jnp.float32),
        pltpu.VMEM((2, FT, 2, 8, 129), jnp.float32),
        pltpu.SemaphoreType.DMA((2,)),
        pltpu.SemaphoreType.DMA((2,)),
    ],
    compiler_params=pltpu.CompilerParams(
        use_tc_tiling_on_sc=False, needs_layout_passes=False
    ),
)
def _embed_kernel(xt_hbm, table_hbm, out_hbm, idx_v, gbuf, tbuf, gsem, ssem):
    cid = lax.axis_index("c")
    sid = lax.axis_index("s")
    wid = sid * NC + cid
    b0 = wid * BPW

    for p in range(NP):
        pltpu.sync_copy(xt_hbm.at[p, pl.ds(b0, BPW)], idx_v.at[p])

    iota = lax.iota(jnp.int32, 16)

    def gathers(s, b, start):
        p = s // 2
        h = s % 2
        for jj in range(2):
            idxref = idx_v.at[p, pl.ds(h * HALF + jj * 128, 128)]
            cp = (pltpu.async_copy if start else pltpu.make_async_copy)(
                table_hbm.at[idxref],
                gbuf.at[b, pl.ds(jj * 128, 128)],
                gsem.at[b],
            )
            if not start:
                cp.wait()

    def scatters(s, b, start):
        p = s // 2
        h = s % 2
        bt0 = wid * BTW + h * 2
        for ft in range(FT):
            for bl in range(2):
                cp = (pltpu.async_copy if start else pltpu.make_async_copy)(
                    tbuf.at[b, ft, bl, :, pl.ds(0, 128)],
                    out_hbm.at[p, ft, bt0 + bl],
                    ssem.at[b],
                )
                if not start:
                    cp.wait()

    zeros16 = jnp.full((16,), 0, jnp.int32)
    ftv = [(iota + j * 16) // 8 for j in range(4)]
    fiv = [(iota + j * 16) % 8 for j in range(4)]

    def transpose_scale(b):
        @plsc.parallel_loop(0, HALF, unroll=2)
        def _r(r):
            btv = zeros16 + r // 128
            biv = zeros16 + r % 128
            for j in range(4):
                vals = gbuf[b, r, pl.ds(j * 16, 16)] * SCALE
                plsc.store_scatter(tbuf.at[b], [ftv[j], btv, fiv[j], biv], vals)

    def step(s, b, first, last):
        gathers(s, b, start=False)
        if not last:
            gathers(s + 1, 1 - b, start=True)
        if not first:
            scatters(s - 2, b, start=False)
        transpose_scale(b)
        scatters(s, b, start=True)

    gathers(0, 0, start=True)
    step(0, 0, first=True, last=False)
    step(1, 1, first=True, last=False)

    @pl.loop(2, NSTEP - 2, step=2)
    def _steady(s):
        step(s, 0, first=False, last=False)
        step(s + 1, 1, first=False, last=False)

    step(NSTEP - 2, 0, first=False, last=False)
    step(NSTEP - 1, 1, first=False, last=True)
    scatters(NSTEP - 2, 0, start=False)
    scatters(NSTEP - 1, 1, start=False)


def kernel(x, table):
    xt = jnp.transpose(x.astype(jnp.int32))
    out5d = _embed_kernel(xt, table)
    return out5d.transpose(2, 4, 0, 1, 3).reshape(NB, NP, D)
